# baseline (speedup 1.0000x reference)
import jax
import jax.numpy as jnp
from jax import lax
from jax.experimental import pallas as pl
from jax.experimental.pallas import tpu as pltpu, tpu_sc as plsc

N_ROWS, N_COLS = 1024, 32768
NUM_WORKERS = 32
ROWS_PER_W = N_ROWS // NUM_WORKERS
R = 16
W = 1024
NCT = N_COLS // W
NRG = ROWS_PER_W // R
U = NRG * NCT
L = 16
IL = 16


def _sc_body(x_hbm, m_hbm, out_hbm,
             xva, xvb, mva, mvb, ova, ovb, cv,
             sxa, sxb, sma, smb, soa, sob):
    wid = lax.axis_index("s") * 2 + lax.axis_index("c")
    idx_last = jnp.full((L, 1), L - 1, dtype=jnp.int32)
    dnums = lax.GatherDimensionNumbers(
        offset_dims=(), collapsed_slice_dims=(0,), start_index_map=(0,))

    def bcast_last(o):
        return lax.gather(o, idx_last, dnums, slice_sizes=(1,),
                          mode=lax.GatherScatterMode.PROMISE_IN_BOUNDS)

    zeros = jnp.zeros((L,), dtype=jnp.float32)

    def coords(u):
        r0 = wid * ROWS_PER_W + (u // NCT) * R
        c0 = (u % NCT) * W
        return r0, c0

    def x_slice(u):
        r0, c0 = coords(u)
        return x_hbm.at[pl.ds(r0, R), pl.ds(c0, W)]

    def m_slice(u):
        r0, c0 = coords(u)
        return m_hbm.at[pl.ds(r0, R), pl.ds(c0, W)]

    def o_slice(u):
        r0, c0 = coords(u)
        return out_hbm.at[pl.ds(r0, R), pl.ds(c0, W)]

    bufs = ((xva, mva, ova, sxa, sma, soa),
            (xvb, mvb, ovb, sxb, smb, sob))

    pltpu.async_copy(x_slice(0), xva, sxa)
    pltpu.async_copy(m_slice(0), mva, sma)

    def pair_body(p, dummy):
        for b in range(2):
            xt, mt, ot, sx, sm, so = bufs[b]
            xn, mn, _, sxn, smn, _ = bufs[1 - b]
            u = p * 2 + b
            un = jnp.minimum(u + 1, U - 1)

            pltpu.async_copy(x_slice(un), xn, sxn)
            pltpu.async_copy(m_slice(un), mn, smn)

            pltpu.make_async_copy(x_slice(u), xt, sx).wait()
            pltpu.make_async_copy(m_slice(u), mt, sm).wait()

            @pl.when(p >= 1)
            def _wait_out():
                pltpu.make_async_copy(ot, o_slice(u - 2), so).wait()

            @pl.when(u % NCT == 0)
            def _reset():
                for r in range(R):
                    cv[r, :] = zeros

            for rb in range(0, R, IL):
                @plsc.parallel_loop(
                    0, W // L, 1, unroll=1,
                    carry=tuple(cv[rb + i, :] for i in range(IL)))
                def vec_body(v, carries):
                    new = []
                    for i in range(IL):
                        r = rb + i
                        sl = pl.ds(v * L, L)
                        xm = xt[r, sl] * mt[r, sl]
                        o = plsc.cumsum(xm) + carries[i]
                        ot[r, sl] = o
                        new.append(bcast_last(o))
                    return tuple(new)

                for i in range(IL):
                    cv[rb + i, :] = vec_body[i]

            pltpu.async_copy(ot, o_slice(u), so)
        return dummy

    lax.fori_loop(0, U // 2, pair_body, 0)

    pltpu.make_async_copy(ova, o_slice(U - 2), soa).wait()
    pltpu.make_async_copy(ovb, o_slice(U - 1), sob).wait()
    pltpu.make_async_copy(x_slice(U - 1), xva, sxa).wait()
    pltpu.make_async_copy(m_slice(U - 1), mva, sma).wait()


def kernel(x, mask):
    m = mask.astype(jnp.float32)
    mesh = plsc.VectorSubcoreMesh(core_axis_name="c", subcore_axis_name="s")
    f = pl.kernel(
        _sc_body,
        out_type=jax.ShapeDtypeStruct((N_ROWS, N_COLS), jnp.float32),
        mesh=mesh,
        compiler_params=pltpu.CompilerParams(needs_layout_passes=False),
        scratch_types=[
            pltpu.VMEM((R, W), jnp.float32),
            pltpu.VMEM((R, W), jnp.float32),
            pltpu.VMEM((R, W), jnp.float32),
            pltpu.VMEM((R, W), jnp.float32),
            pltpu.VMEM((R, W), jnp.float32),
            pltpu.VMEM((R, W), jnp.float32),
            pltpu.VMEM((R, L), jnp.float32),
            pltpu.SemaphoreType.DMA,
            pltpu.SemaphoreType.DMA,
            pltpu.SemaphoreType.DMA,
            pltpu.SemaphoreType.DMA,
            pltpu.SemaphoreType.DMA,
            pltpu.SemaphoreType.DMA,
        ],
    )
    return f(x, m)


# speedup vs baseline: 1.0018x; 1.0018x over previous
import jax
import jax.numpy as jnp
from jax import lax
from jax.experimental import pallas as pl
from jax.experimental.pallas import tpu as pltpu, tpu_sc as plsc

N_ROWS, N_COLS = 1024, 32768
NUM_WORKERS = 32
ROWS_PER_W = N_ROWS // NUM_WORKERS
R = 16
W = 1024
NCT = N_COLS // W
NRG = ROWS_PER_W // R
U = NRG * NCT
L = 16
IL = 8


def _sc_body(x_hbm, m_hbm, out_hbm,
             xva, xvb, mva, mvb, ova, ovb, cv,
             sxa, sxb, sma, smb, soa, sob):
    wid = lax.axis_index("s") * 2 + lax.axis_index("c")
    idx_last = jnp.full((L, 1), L - 1, dtype=jnp.int32)
    dnums = lax.GatherDimensionNumbers(
        offset_dims=(), collapsed_slice_dims=(0,), start_index_map=(0,))

    def bcast_last(o):
        return jnp.broadcast_to(o[L - 1], (L,))

    zeros = jnp.zeros((L,), dtype=jnp.float32)

    def coords(u):
        r0 = wid * ROWS_PER_W + (u // NCT) * R
        c0 = (u % NCT) * W
        return r0, c0

    def x_slice(u):
        r0, c0 = coords(u)
        return x_hbm.at[pl.ds(r0, R), pl.ds(c0, W)]

    def m_slice(u):
        r0, c0 = coords(u)
        return m_hbm.at[pl.ds(r0, R), pl.ds(c0, W)]

    def o_slice(u):
        r0, c0 = coords(u)
        return out_hbm.at[pl.ds(r0, R), pl.ds(c0, W)]

    bufs = ((xva, mva, ova, sxa, sma, soa),
            (xvb, mvb, ovb, sxb, smb, sob))

    pltpu.async_copy(x_slice(0), xva, sxa)
    pltpu.async_copy(m_slice(0), mva, sma)

    def pair_body(p, dummy):
        for b in range(2):
            xt, mt, ot, sx, sm, so = bufs[b]
            xn, mn, _, sxn, smn, _ = bufs[1 - b]
            u = p * 2 + b
            un = jnp.minimum(u + 1, U - 1)

            pltpu.async_copy(x_slice(un), xn, sxn)
            pltpu.async_copy(m_slice(un), mn, smn)

            pltpu.make_async_copy(x_slice(u), xt, sx).wait()
            pltpu.make_async_copy(m_slice(u), mt, sm).wait()

            @pl.when(p >= 1)
            def _wait_out():
                pltpu.make_async_copy(ot, o_slice(u - 2), so).wait()

            @pl.when(u % NCT == 0)
            def _reset():
                for r in range(R):
                    cv[r, :] = zeros

            for rb in range(0, R, IL):
                @plsc.parallel_loop(
                    0, W // L, 1, unroll=2,
                    carry=tuple(cv[rb + i, :] for i in range(IL)))
                def vec_body(v, carries):
                    new = []
                    for i in range(IL):
                        r = rb + i
                        sl = pl.ds(v * L, L)
                        xm = xt[r, sl] * mt[r, sl]
                        o = plsc.cumsum(xm) + carries[i]
                        ot[r, sl] = o
                        new.append(bcast_last(o))
                    return tuple(new)

                for i in range(IL):
                    cv[rb + i, :] = vec_body[i]

            pltpu.async_copy(ot, o_slice(u), so)
        return dummy

    lax.fori_loop(0, U // 2, pair_body, 0)

    pltpu.make_async_copy(ova, o_slice(U - 2), soa).wait()
    pltpu.make_async_copy(ovb, o_slice(U - 1), sob).wait()
    pltpu.make_async_copy(x_slice(U - 1), xva, sxa).wait()
    pltpu.make_async_copy(m_slice(U - 1), mva, sma).wait()


def kernel(x, mask):
    m = mask.astype(jnp.float32)
    mesh = plsc.VectorSubcoreMesh(core_axis_name="c", subcore_axis_name="s")
    f = pl.kernel(
        _sc_body,
        out_type=jax.ShapeDtypeStruct((N_ROWS, N_COLS), jnp.float32),
        mesh=mesh,
        compiler_params=pltpu.CompilerParams(needs_layout_passes=False),
        scratch_types=[
            pltpu.VMEM((R, W), jnp.float32),
            pltpu.VMEM((R, W), jnp.float32),
            pltpu.VMEM((R, W), jnp.float32),
            pltpu.VMEM((R, W), jnp.float32),
            pltpu.VMEM((R, W), jnp.float32),
            pltpu.VMEM((R, W), jnp.float32),
            pltpu.VMEM((R, L), jnp.float32),
            pltpu.SemaphoreType.DMA,
            pltpu.SemaphoreType.DMA,
            pltpu.SemaphoreType.DMA,
            pltpu.SemaphoreType.DMA,
            pltpu.SemaphoreType.DMA,
            pltpu.SemaphoreType.DMA,
        ],
    )
    return f(x, m)
